# Initial kernel scaffold; baseline (speedup 1.0000x reference)
import jax
import jax.numpy as jnp
from jax.experimental import pallas as pl


def kernel(user_idx, item_idx, ue_gmf, ie_gmf, ue_mlp, ie_mlp, W1, b1, Wo, bo):
    raise NotImplementedError("write your pallas kernel here")


# speedup vs baseline: 7.3961x; 7.3961x over previous
import functools

import jax
import jax.numpy as jnp
from jax import lax
from jax.experimental import pallas as pl
from jax.experimental.pallas import tpu as pltpu
from jax.experimental.pallas import tpu_sc as plsc

B = 16384
D = 128
NC = 2
NS = 16
NW = NC * NS
B_PER_W = B // NW
CH = 128
N_CH = B_PER_W // CH


def _sc_gather_body(uidx_hbm, iidx_hbm, ue_gmf, ie_gmf, ue_mlp, ie_mlp,
                    o_ug, o_ig, o_um, o_im,
                    uix, iix, bug, big, bum, bim, sem):
    wid = lax.axis_index("s") * NC + lax.axis_index("c")
    base = wid * B_PER_W
    for c in range(N_CH):
        off = base + c * CH
        pltpu.sync_copy(uidx_hbm.at[pl.ds(off, CH)], uix)
        pltpu.sync_copy(iidx_hbm.at[pl.ds(off, CH)], iix)
        cps = [
            pltpu.async_copy(ue_gmf.at[uix], bug, sem),
            pltpu.async_copy(ie_gmf.at[iix], big, sem),
            pltpu.async_copy(ue_mlp.at[uix], bum, sem),
            pltpu.async_copy(ie_mlp.at[iix], bim, sem),
        ]
        for cp in cps:
            cp.wait()
        pltpu.sync_copy(bug, o_ug.at[pl.ds(off, CH)])
        pltpu.sync_copy(big, o_ig.at[pl.ds(off, CH)])
        pltpu.sync_copy(bum, o_um.at[pl.ds(off, CH)])
        pltpu.sync_copy(bim, o_im.at[pl.ds(off, CH)])


_sc_gather = functools.partial(
    pl.kernel,
    mesh=plsc.VectorSubcoreMesh(core_axis_name="c", subcore_axis_name="s"),
    out_type=tuple(jax.ShapeDtypeStruct((B, D), jnp.float32) for _ in range(4)),
    scratch_types=[
        pltpu.VMEM((CH,), jnp.int32),
        pltpu.VMEM((CH,), jnp.int32),
        pltpu.VMEM((CH, D), jnp.float32),
        pltpu.VMEM((CH, D), jnp.float32),
        pltpu.VMEM((CH, D), jnp.float32),
        pltpu.VMEM((CH, D), jnp.float32),
        pltpu.SemaphoreType.DMA,
    ],
)(_sc_gather_body)


TC_BLK = 2048


def _tc_body(ug_r, ig_r, um_r, im_r, w1_r, b1_r, wo_r, bo_r, out_r):
    gmf = ug_r[...] * ig_r[...]
    h = jnp.dot(um_r[...], w1_r[0:D, :], preferred_element_type=jnp.float32)
    h = h + jnp.dot(im_r[...], w1_r[D:2 * D, :], preferred_element_type=jnp.float32)
    h = jnp.maximum(h + b1_r[...], 0.0)
    wo = wo_r[...]
    out = jnp.sum(gmf * wo[:, :D], axis=1, keepdims=True)
    out = out + jnp.sum(h * wo[:, D:], axis=1, keepdims=True)
    out_r[...] = out + bo_r[...]


def _tc_forward(ug, ig, um, im, W1, b1_row, wo_row, bo_11):
    grid = (B // TC_BLK,)
    blk = lambda i: (i, 0)
    whole = lambda i: (0, 0)
    return pl.pallas_call(
        _tc_body,
        grid=grid,
        in_specs=[
            pl.BlockSpec((TC_BLK, D), blk),
            pl.BlockSpec((TC_BLK, D), blk),
            pl.BlockSpec((TC_BLK, D), blk),
            pl.BlockSpec((TC_BLK, D), blk),
            pl.BlockSpec((2 * D, D), whole),
            pl.BlockSpec((1, D), whole),
            pl.BlockSpec((1, 2 * D), whole),
            pl.BlockSpec((1, 1), whole),
        ],
        out_specs=pl.BlockSpec((TC_BLK, 1), blk),
        out_shape=jax.ShapeDtypeStruct((B, 1), jnp.float32),
    )(ug, ig, um, im, W1, b1_row, wo_row, bo_11)


def kernel(user_idx, item_idx, ue_gmf, ie_gmf, ue_mlp, ie_mlp, W1, b1, Wo, bo):
    ug, ig, um, im = _sc_gather(user_idx, item_idx, ue_gmf, ie_gmf, ue_mlp, ie_mlp)
    return _tc_forward(ug, ig, um, im, W1,
                       b1.reshape(1, D), Wo.reshape(1, 2 * D), bo.reshape(1, 1))


# speedup vs baseline: 7.6496x; 1.0343x over previous
import functools

import jax
import jax.numpy as jnp
from jax import lax
from jax.experimental import pallas as pl
from jax.experimental.pallas import tpu as pltpu
from jax.experimental.pallas import tpu_sc as plsc

B = 16384
D = 128
NC = 2
NS = 16
NW = NC * NS
B_PER_W = B // NW
CH = 64
N_CH = B_PER_W // CH
NBUF = 3
VPR = D // 16


def _sc_gather_body(uidx_hbm, iidx_hbm, ue_gmf, ie_gmf, ue_mlp, ie_mlp,
                    o_gmf, o_um, o_im,
                    uix, iix, bufs, gsem, wsem):
    wid = lax.axis_index("s") * NC + lax.axis_index("c")
    base = wid * B_PER_W

    for c in range(N_CH):
        pltpu.sync_copy(uidx_hbm.at[pl.ds(base + c * CH, CH)], uix.at[c])
        pltpu.sync_copy(iidx_hbm.at[pl.ds(base + c * CH, CH)], iix.at[c])

    def gather(c, s):
        bug, big, bum, bim = bufs[s]
        return [
            pltpu.async_copy(ue_gmf.at[uix.at[c]], bug, gsem),
            pltpu.async_copy(ie_gmf.at[iix.at[c]], big, gsem),
            pltpu.async_copy(ue_mlp.at[uix.at[c]], bum, gsem),
            pltpu.async_copy(ie_mlp.at[iix.at[c]], bim, gsem),
        ]

    def gmf_mul(s):
        bug, big = bufs[s][0], bufs[s][1]

        def row(r, carry):
            for j in range(VPR):
                sl = pl.ds(j * 16, 16)
                bug[r, sl] = bug[r, sl] * big[r, sl]
            return carry

        lax.fori_loop(0, CH, row, 0)

    def writeback(c, s):
        bug, _, bum, bim = bufs[s]
        rows = pl.ds(base + c * CH, CH)
        return [
            pltpu.async_copy(bug, o_gmf.at[rows], wsem),
            pltpu.async_copy(bum, o_um.at[rows], wsem),
            pltpu.async_copy(bim, o_im.at[rows], wsem),
        ]

    g = {}
    wb = {}
    for c in range(min(2, N_CH)):
        g[c] = gather(c, c % NBUF)
    for c in range(N_CH):
        s = c % NBUF
        for d in g.pop(c):
            d.wait()
        gmf_mul(s)
        wb[s] = writeback(c, s)
        nc = c + 2
        if nc < N_CH:
            ns = nc % NBUF
            if ns in wb:
                for d in wb.pop(ns):
                    d.wait()
            g[nc] = gather(nc, ns)
    for s in list(wb):
        for d in wb.pop(s):
            d.wait()


_sc_gather = functools.partial(
    pl.kernel,
    mesh=plsc.VectorSubcoreMesh(core_axis_name="c", subcore_axis_name="s"),
    out_type=tuple(jax.ShapeDtypeStruct((B, D), jnp.float32) for _ in range(3)),
    scratch_types=[
        pltpu.VMEM((N_CH, CH), jnp.int32),
        pltpu.VMEM((N_CH, CH), jnp.int32),
        tuple(tuple(pltpu.VMEM((CH, D), jnp.float32) for _ in range(4))
              for _ in range(NBUF)),
        pltpu.SemaphoreType.DMA,
        pltpu.SemaphoreType.DMA,
    ],
)(_sc_gather_body)


TC_BLK = 2048


def _tc_body(gmf_r, um_r, im_r, w1_r, b1_r, wo_r, bo_r, out_r):
    h = jnp.dot(um_r[...], w1_r[0:D, :], preferred_element_type=jnp.float32)
    h = h + jnp.dot(im_r[...], w1_r[D:2 * D, :], preferred_element_type=jnp.float32)
    h = jnp.maximum(h + b1_r[...], 0.0)
    wo = wo_r[...]
    out = jnp.sum(gmf_r[...] * wo[:, :D], axis=1, keepdims=True)
    out = out + jnp.sum(h * wo[:, D:], axis=1, keepdims=True)
    out_r[...] = out + bo_r[...]


def _tc_forward(gmf, um, im, W1, b1_row, wo_row, bo_11):
    grid = (B // TC_BLK,)
    blk = lambda i: (i, 0)
    whole = lambda i: (0, 0)
    return pl.pallas_call(
        _tc_body,
        grid=grid,
        in_specs=[
            pl.BlockSpec((TC_BLK, D), blk),
            pl.BlockSpec((TC_BLK, D), blk),
            pl.BlockSpec((TC_BLK, D), blk),
            pl.BlockSpec((2 * D, D), whole),
            pl.BlockSpec((1, D), whole),
            pl.BlockSpec((1, 2 * D), whole),
            pl.BlockSpec((1, 1), whole),
        ],
        out_specs=pl.BlockSpec((TC_BLK, 1), blk),
        out_shape=jax.ShapeDtypeStruct((B, 1), jnp.float32),
    )(gmf, um, im, W1, b1_row, wo_row, bo_11)


def kernel(user_idx, item_idx, ue_gmf, ie_gmf, ue_mlp, ie_mlp, W1, b1, Wo, bo):
    gmf, um, im = _sc_gather(user_idx, item_idx, ue_gmf, ie_gmf, ue_mlp, ie_mlp)
    return _tc_forward(gmf, um, im, W1,
                       b1.reshape(1, D), Wo.reshape(1, 2 * D), bo.reshape(1, 1))


# speedup vs baseline: 8.5690x; 1.1202x over previous
import functools

import jax
import jax.numpy as jnp
from jax import lax
from jax.experimental import pallas as pl
from jax.experimental.pallas import tpu as pltpu
from jax.experimental.pallas import tpu_sc as plsc

B = 16384
D = 128
NC = 2
NS = 16
NW = NC * NS
B_PER_W = B // NW
CH = 64
N_CH = B_PER_W // CH
NBUF = 3
VPR = D // 16


def _sc_gather_body(uidx_hbm, iidx_hbm, ue_gmf, ie_gmf, ue_mlp, ie_mlp,
                    o_gmf, o_um, o_im,
                    uix, iix, bufs, gsem, wsem):
    wid = lax.axis_index("s") * NC + lax.axis_index("c")
    base = wid * B_PER_W

    for c in range(N_CH):
        pltpu.sync_copy(uidx_hbm.at[pl.ds(base + c * CH, CH)], uix.at[c])
        pltpu.sync_copy(iidx_hbm.at[pl.ds(base + c * CH, CH)], iix.at[c])

    def gather(c, s):
        bug, big, bum, bim = bufs[s]
        return [
            pltpu.async_copy(ue_gmf.at[uix.at[c]], bug, gsem),
            pltpu.async_copy(ie_gmf.at[iix.at[c]], big, gsem),
            pltpu.async_copy(ue_mlp.at[uix.at[c]], bum, gsem),
            pltpu.async_copy(ie_mlp.at[iix.at[c]], bim, gsem),
        ]

    def gmf_mul(s):
        bug, big = bufs[s][0], bufs[s][1]

        def row(r, carry):
            for j in range(VPR):
                sl = pl.ds(j * 16, 16)
                bug[r, sl] = bug[r, sl] * big[r, sl]
            return carry

        lax.fori_loop(0, CH, row, 0)

    def writeback(c, s):
        bug, _, bum, bim = bufs[s]
        rows = pl.ds(base + c * CH, CH)
        return [
            pltpu.async_copy(bug, o_gmf.at[rows], wsem),
            pltpu.async_copy(bum, o_um.at[rows], wsem),
            pltpu.async_copy(bim, o_im.at[rows], wsem),
        ]

    g = {}
    wb = {}
    for c in range(min(2, N_CH)):
        g[c] = gather(c, c % NBUF)
    for c in range(N_CH):
        s = c % NBUF
        for d in g.pop(c):
            d.wait()
        gmf_mul(s)
        wb[s] = writeback(c, s)
        nc = c + 2
        if nc < N_CH:
            ns = nc % NBUF
            if ns in wb:
                for d in wb.pop(ns):
                    d.wait()
            g[nc] = gather(nc, ns)
    for s in list(wb):
        for d in wb.pop(s):
            d.wait()


_sc_gather = functools.partial(
    pl.kernel,
    mesh=plsc.VectorSubcoreMesh(core_axis_name="c", subcore_axis_name="s"),
    out_type=tuple(jax.ShapeDtypeStruct((B, D), jnp.float32) for _ in range(3)),
    scratch_types=[
        pltpu.VMEM((N_CH, CH), jnp.int32),
        pltpu.VMEM((N_CH, CH), jnp.int32),
        tuple(tuple(pltpu.VMEM((CH, D), jnp.float32) for _ in range(4))
              for _ in range(NBUF)),
        pltpu.SemaphoreType.DMA,
        pltpu.SemaphoreType.DMA,
    ],
)(_sc_gather_body)


TC_BLK = 2048

_CONTRACT_01 = (((0,), (1,)), ((), ()))
_CONTRACT_11 = (((1,), (1,)), ((), ()))


def _tc_body(gmf_r, um_r, im_r, w1_r, b1_r, wo_r, bo_r, out_r):
    w1 = w1_r[...]
    h_t = lax.dot_general(w1[0:D, :], um_r[...], _CONTRACT_01,
                          preferred_element_type=jnp.float32)
    h_t = h_t + lax.dot_general(w1[D:2 * D, :], im_r[...], _CONTRACT_01,
                                preferred_element_type=jnp.float32)
    h_t = jnp.maximum(h_t + b1_r[...], 0.0)
    wo = wo_r[...]
    out = lax.dot_general(wo[:, :D], gmf_r[...], _CONTRACT_11,
                          preferred_element_type=jnp.float32)
    out = out + jnp.dot(wo[:, D:], h_t, preferred_element_type=jnp.float32)
    out_r[...] = out + bo_r[...]


def _tc_forward(gmf, um, im, W1, b1_col, wo_row, bo_11):
    grid = (B // TC_BLK,)
    blk = lambda i: (i, 0)
    lane_blk = lambda i: (0, i)
    whole = lambda i: (0, 0)
    out = pl.pallas_call(
        _tc_body,
        grid=grid,
        in_specs=[
            pl.BlockSpec((TC_BLK, D), blk),
            pl.BlockSpec((TC_BLK, D), blk),
            pl.BlockSpec((TC_BLK, D), blk),
            pl.BlockSpec((2 * D, D), whole),
            pl.BlockSpec((D, 1), whole),
            pl.BlockSpec((1, 2 * D), whole),
            pl.BlockSpec((1, 1), whole),
        ],
        out_specs=pl.BlockSpec((1, TC_BLK), lane_blk),
        out_shape=jax.ShapeDtypeStruct((1, B), jnp.float32),
    )(gmf, um, im, W1, b1_col, wo_row, bo_11)
    return out.reshape(B, 1)


def kernel(user_idx, item_idx, ue_gmf, ie_gmf, ue_mlp, ie_mlp, W1, b1, Wo, bo):
    gmf, um, im = _sc_gather(user_idx, item_idx, ue_gmf, ie_gmf, ue_mlp, ie_mlp)
    return _tc_forward(gmf, um, im, W1,
                       b1.reshape(D, 1), Wo.reshape(1, 2 * D), bo.reshape(1, 1))
